# baseline (speedup 1.0000x reference)
import functools

import jax
import jax.numpy as jnp
from jax import lax
from jax.experimental import pallas as pl
from jax.experimental.pallas import tpu as pltpu
from jax.experimental.pallas import tpu_sc as plsc

_N = 8192
_K = 81
_BLK = 256
_C = 128
_BIS = 12


def _sim_thr_body(xnb_ref, xnt_ref, sim_ref, thr_ref):
    b, n = sim_ref.shape
    row0 = pl.program_id(0) * b
    col_ids = lax.broadcasted_iota(jnp.int32, (b, n), 1)
    row_ids = row0 + lax.broadcasted_iota(jnp.int32, (b, n), 0)
    xb16 = xnb_ref[:, :].astype(jnp.bfloat16)
    yt16 = xnt_ref[:, :].astype(jnp.bfloat16)
    sim = jnp.dot(xb16, yt16, preferred_element_type=jnp.float32)
    sim = jnp.where(col_ids == row_ids, jnp.float32(2.0), sim)
    sim_ref[:, :] = sim

    lo0 = jnp.full((b, 1), -1.0, jnp.float32)
    hi0 = jnp.full((b, 1), 1.0, jnp.float32)

    def bis(_, c):
        lo, hi = c
        mid = jnp.float32(0.5) * (lo + hi)
        cnt = jnp.sum(jnp.where(sim_ref[:, :] >= mid, 1.0, 0.0),
                      axis=1, keepdims=True)
        p = cnt >= jnp.float32(_K)
        return (jnp.where(p, mid, lo), jnp.where(p, hi, mid))

    lo, hi = lax.fori_loop(0, _BIS, bis, (lo0, hi0))
    thr_ref[:, :] = jnp.broadcast_to(lo, (b, 16))


def _sim_thr_call(xn, xnt):
    n = xn.shape[0]
    grid = n // _BLK
    return pl.pallas_call(
        _sim_thr_body,
        grid=(grid,),
        in_specs=[
            pl.BlockSpec((_BLK, 3), lambda i: (i, 0)),
            pl.BlockSpec((3, n), lambda i: (0, 0)),
        ],
        out_specs=[
            pl.BlockSpec((_BLK, n), lambda i: (i, 0)),
            pl.BlockSpec((_BLK, 16), lambda i: (i, 0)),
        ],
        out_shape=[
            jax.ShapeDtypeStruct((n, n), jnp.float32),
            jax.ShapeDtypeStruct((n, 16), jnp.float32),
        ],
    )(xn, xnt)


def _make_sc_compact(n):
    info = plsc.get_sparse_core_info()
    nw = info.num_cores * info.num_subcores
    rows_w = n // nw
    rb_rows = 64
    nvec = n // 16
    i32, f32 = jnp.int32, jnp.float32
    mesh = plsc.VectorSubcoreMesh(core_axis_name="c", subcore_axis_name="s")

    @functools.partial(
        pl.kernel,
        mesh=mesh,
        out_type=(jax.ShapeDtypeStruct((n * _C,), f32),) * 4,
        compiler_params=pltpu.CompilerParams(needs_layout_passes=False),
        scratch_types=[
            pltpu.VMEM((n,), f32),
            pltpu.VMEM((n,), f32),
            pltpu.VMEM((n,), f32),
            pltpu.VMEM((n,), f32),
            pltpu.VMEM((n,), f32),
            pltpu.VMEM((rows_w * 16,), f32),
            pltpu.VMEM((144,), i32),
            pltpu.SemaphoreType.DMA,
            pltpu.SemaphoreType.DMA,
            pltpu.VMEM((rb_rows * _C,), f32),
            pltpu.VMEM((rb_rows * _C,), f32),
            pltpu.VMEM((rb_rows * _C,), f32),
            pltpu.VMEM((rb_rows * _C,), f32),
        ],
    )
    def sc_compact(sim_hbm, thr_hbm, x_hbm, y_hbm, z_hbm,
                   cv_hbm, cx_hbm, cy_hbm, cz_hbm,
                   xv, yv, zv, rowbuf0, rowbuf1, thrv, ci, sem0, sem1,
                   ov, ox, oy, oz):
        wid = lax.axis_index("s") * info.num_cores + lax.axis_index("c")
        r0 = wid * rows_w
        pltpu.sync_copy(x_hbm, xv)
        pltpu.sync_copy(y_hbm, yv)
        pltpu.sync_copy(z_hbm, zv)
        pltpu.sync_copy(thr_hbm.at[pl.ds(r0 * 16, rows_w * 16)], thrv)
        iota16 = lax.iota(i32, 16)
        zero16 = jnp.zeros((16,), i32)

        def process(rowbuf, r, bb):
            tv = thrv[pl.ds((bb * rb_rows + r) * 16, 16)]

            def scan(vb, ptr_v):
                o = vb * 16
                s = rowbuf[pl.ds(o, 16)]
                msk = s >= tv
                pos = ptr_v + plsc.cumsum(msk.astype(i32)) - 1
                pos = jnp.minimum(pos, jnp.int32(136))
                plsc.store_scatter(ci, [pos], iota16 + o, mask=msk)
                return ptr_v + plsc.all_reduce_population_count(msk)

            cnt16 = lax.fori_loop(0, nvec, scan, zero16, unroll=8)
            for t in range(_C // 16):
                ii = ci[pl.ds(t * 16, 16)]
                valid = (iota16 + t * 16) < cnt16
                vals = plsc.load_gather(rowbuf, [ii], mask=valid)
                ob = r * _C + t * 16
                ov[pl.ds(ob, 16)] = jnp.where(
                    valid, vals, jnp.float32(-3.0))
                ox[pl.ds(ob, 16)] = plsc.load_gather(xv, [ii], mask=valid)
                oy[pl.ds(ob, 16)] = plsc.load_gather(yv, [ii], mask=valid)
                oz[pl.ds(ob, 16)] = plsc.load_gather(zv, [ii], mask=valid)

        def start(buf, sem, g):
            pltpu.async_copy(sim_hbm.at[pl.ds(g * n, n)], buf, sem)

        def drain(buf, sem):
            pltpu.make_async_copy(sim_hbm.at[pl.ds(0, n)], buf, sem).wait()

        for bb in range(rows_w // rb_rows):
            g0 = r0 + bb * rb_rows
            start(rowbuf0, sem0, g0)

            def pair(p, _, bb=bb, g0=g0):
                r = 2 * p
                drain(rowbuf0, sem0)
                start(rowbuf1, sem1, g0 + r + 1)
                process(rowbuf0, r, bb)
                drain(rowbuf1, sem1)
                start(rowbuf0, sem0,
                      g0 + jnp.minimum(r + 2, rb_rows - 1))
                process(rowbuf1, r + 1, bb)
                return 0

            lax.fori_loop(0, rb_rows // 2, pair, 0)
            drain(rowbuf0, sem0)
            base = g0 * _C
            pltpu.sync_copy(ov, cv_hbm.at[pl.ds(base, rb_rows * _C)])
            pltpu.sync_copy(ox, cx_hbm.at[pl.ds(base, rb_rows * _C)])
            pltpu.sync_copy(oy, cy_hbm.at[pl.ds(base, rb_rows * _C)])
            pltpu.sync_copy(oz, cz_hbm.at[pl.ds(base, rb_rows * _C)])

    return sc_compact


def _rank_body(cv_ref, cx_ref, cy_ref, cz_ref, pts_ref, dwt_ref, w_ref,
               bias_ref, out_ref, cv_s, sx_s, sy_s, sz_s):
    f32 = jnp.float32
    b = cv_ref.shape[0]
    lane_c = lax.broadcasted_iota(jnp.int32, (b, _C), 1)
    lane_k = lax.broadcasted_iota(jnp.int32, (b, _K), 1)
    cv_s[:, :] = cv_ref[:, :]
    cx = cx_ref[:, :]
    cy = cy_ref[:, :]
    cz = cz_ref[:, :]
    sx_s[:, :] = jnp.zeros((b, _K), f32)
    sy_s[:, :] = jnp.zeros((b, _K), f32)
    sz_s[:, :] = jnp.zeros((b, _K), f32)

    def body(j, _):
        cv = cv_s[:, :]
        m = jnp.max(cv, axis=1, keepdims=True)
        a = jnp.min(jnp.where(cv == m, lane_c, jnp.int32(_C)),
                    axis=1, keepdims=True)
        oh = lane_c == a
        vx = jnp.sum(jnp.where(oh, cx, 0.0), axis=1, keepdims=True)
        vy = jnp.sum(jnp.where(oh, cy, 0.0), axis=1, keepdims=True)
        vz = jnp.sum(jnp.where(oh, cz, 0.0), axis=1, keepdims=True)
        cv_s[:, :] = jnp.where(oh, jnp.float32(-3.0), cv)
        kj = lane_k == j
        sx_s[:, :] = jnp.where(kj, vx, sx_s[:, :])
        sy_s[:, :] = jnp.where(kj, vy, sy_s[:, :])
        sz_s[:, :] = jnp.where(kj, vz, sz_s[:, :])
        return 0

    lax.fori_loop(0, _K, body, 0)

    gx = sx_s[:, :]
    gy = sy_s[:, :]
    gz = sz_s[:, :]
    dx = gx - pts_ref[:, 0:1]
    dy = gy - pts_ref[:, 1:2]
    dz = gz - pts_ref[:, 2:3]
    dist = dx * dx + dy * dy + dz * dz + jnp.float32(1.0)
    acc = jnp.dot(gx * dwt_ref[0:1, :], w_ref[0], preferred_element_type=f32)
    acc = acc + jnp.dot(gy * dwt_ref[1:2, :], w_ref[1],
                        preferred_element_type=f32)
    acc = acc + jnp.dot(gz * dwt_ref[2:3, :], w_ref[2],
                        preferred_element_type=f32)
    acc = acc + jnp.dot(dist * dwt_ref[3:4, :], w_ref[3],
                        preferred_element_type=f32)
    out_ref[:, :] = acc + bias_ref[:, :]


def _rank_call(cv, cx, cy, cz, points, dwt, weight, bias2d):
    n = points.shape[0]
    grid = n // _BLK
    cout = weight.shape[2]
    return pl.pallas_call(
        _rank_body,
        grid=(grid,),
        in_specs=[
            pl.BlockSpec((_BLK, _C), lambda i: (i, 0)),
            pl.BlockSpec((_BLK, _C), lambda i: (i, 0)),
            pl.BlockSpec((_BLK, _C), lambda i: (i, 0)),
            pl.BlockSpec((_BLK, _C), lambda i: (i, 0)),
            pl.BlockSpec((_BLK, 3), lambda i: (i, 0)),
            pl.BlockSpec((4, _K), lambda i: (0, 0)),
            pl.BlockSpec((4, _K, cout), lambda i: (0, 0, 0)),
            pl.BlockSpec((1, cout), lambda i: (0, 0)),
        ],
        out_specs=pl.BlockSpec((_BLK, cout), lambda i: (i, 0)),
        out_shape=jax.ShapeDtypeStruct((n, cout), jnp.float32),
        scratch_shapes=[
            pltpu.VMEM((_BLK, _C), jnp.float32),
            pltpu.VMEM((_BLK, _K), jnp.float32),
            pltpu.VMEM((_BLK, _K), jnp.float32),
            pltpu.VMEM((_BLK, _K), jnp.float32),
        ],
    )(cv, cx, cy, cz, points, dwt, weight, bias2d)


def kernel(points, dw, weight, bias):
    n = points.shape[0]
    cout = weight.shape[2]
    norm = jnp.linalg.norm(points[:, :3], axis=-1, keepdims=True)
    xn = points[:, :3] / jnp.maximum(norm, 1e-12)
    xnt = xn.T

    sim, thr16 = _sim_thr_call(xn, xnt)

    px = jnp.asarray(points[:, 0], jnp.float32)
    py = jnp.asarray(points[:, 1], jnp.float32)
    pz = jnp.asarray(points[:, 2], jnp.float32)
    cv1, cx1, cy1, cz1 = _make_sc_compact(n)(
        sim.reshape(-1), thr16.reshape(-1), px, py, pz)
    cv = cv1.reshape(n, _C)
    cx = cx1.reshape(n, _C)
    cy = cy1.reshape(n, _C)
    cz = cz1.reshape(n, _C)

    return _rank_call(cv, cx, cy, cz, points, dw.T, weight,
                      bias.reshape(1, cout))


# speedup vs baseline: 1.7023x; 1.7023x over previous
import functools

import jax
import jax.numpy as jnp
from jax import lax
from jax.experimental import pallas as pl
from jax.experimental.pallas import tpu as pltpu
from jax.experimental.pallas import tpu_sc as plsc

_N = 8192
_K = 81
_BLK = 256
_C = 128
_BIS = 12


def _sim_thr_body(xnb_ref, xnt_ref, sim_ref, thr_ref):
    b, n = sim_ref.shape
    row0 = pl.program_id(0) * b
    col_ids = lax.broadcasted_iota(jnp.int32, (b, n), 1)
    row_ids = row0 + lax.broadcasted_iota(jnp.int32, (b, n), 0)
    xb16 = xnb_ref[:, :].astype(jnp.bfloat16)
    yt16 = xnt_ref[:, :].astype(jnp.bfloat16)
    sim = jnp.dot(xb16, yt16, preferred_element_type=jnp.float32)
    sim = jnp.where(col_ids == row_ids, jnp.float32(2.0), sim)
    sim_ref[:, :] = sim

    lo0 = jnp.full((b, 1), -1.0, jnp.float32)
    hi0 = jnp.full((b, 1), 1.0, jnp.float32)

    def bis(_, c):
        lo, hi = c
        mid = jnp.float32(0.5) * (lo + hi)
        cnt = jnp.sum(jnp.where(sim_ref[:, :] >= mid, 1.0, 0.0),
                      axis=1, keepdims=True)
        p = cnt >= jnp.float32(_K)
        return (jnp.where(p, mid, lo), jnp.where(p, hi, mid))

    lo, hi = lax.fori_loop(0, _BIS, bis, (lo0, hi0))
    thr_ref[:, :] = jnp.broadcast_to(lo, (b, 16))


def _sim_thr_call(xn, xnt):
    n = xn.shape[0]
    grid = n // _BLK
    return pl.pallas_call(
        _sim_thr_body,
        grid=(grid,),
        in_specs=[
            pl.BlockSpec((_BLK, 3), lambda i: (i, 0)),
            pl.BlockSpec((3, n), lambda i: (0, 0)),
        ],
        out_specs=[
            pl.BlockSpec((_BLK, n), lambda i: (i, 0)),
            pl.BlockSpec((_BLK, 16), lambda i: (i, 0)),
        ],
        out_shape=[
            jax.ShapeDtypeStruct((n, n), jnp.float32),
            jax.ShapeDtypeStruct((n, 16), jnp.float32),
        ],
    )(xn, xnt)


def _make_sc_compact(n):
    info = plsc.get_sparse_core_info()
    nw = info.num_cores * info.num_subcores
    rows_w = n // nw
    rb_rows = 64
    nvec = n // 16
    i32, f32 = jnp.int32, jnp.float32
    mesh = plsc.VectorSubcoreMesh(core_axis_name="c", subcore_axis_name="s")

    @functools.partial(
        pl.kernel,
        mesh=mesh,
        out_type=(jax.ShapeDtypeStruct((n * _C,), f32),) * 4,
        compiler_params=pltpu.CompilerParams(needs_layout_passes=False),
        scratch_types=[
            pltpu.VMEM((n,), f32),
            pltpu.VMEM((n,), f32),
            pltpu.VMEM((n,), f32),
            pltpu.VMEM((n,), f32),
            pltpu.VMEM((n,), f32),
            pltpu.VMEM((rows_w * 16,), f32),
            pltpu.VMEM((144,), i32),
            pltpu.SemaphoreType.DMA,
            pltpu.SemaphoreType.DMA,
            pltpu.VMEM((rb_rows * _C,), f32),
            pltpu.VMEM((rb_rows * _C,), f32),
            pltpu.VMEM((rb_rows * _C,), f32),
            pltpu.VMEM((rb_rows * _C,), f32),
        ],
    )
    def sc_compact(sim_hbm, thr_hbm, x_hbm, y_hbm, z_hbm,
                   cv_hbm, cx_hbm, cy_hbm, cz_hbm,
                   xv, yv, zv, rowbuf0, rowbuf1, thrv, ci, sem0, sem1,
                   ov, ox, oy, oz):
        wid = lax.axis_index("s") * info.num_cores + lax.axis_index("c")
        r0 = wid * rows_w
        pltpu.sync_copy(x_hbm, xv)
        pltpu.sync_copy(y_hbm, yv)
        pltpu.sync_copy(z_hbm, zv)
        pltpu.sync_copy(thr_hbm.at[pl.ds(r0 * 16, rows_w * 16)], thrv)
        iota16 = lax.iota(i32, 16)
        zero16 = jnp.zeros((16,), i32)

        def process(rowbuf, r, bb):
            tv = thrv[pl.ds((bb * rb_rows + r) * 16, 16)]

            @plsc.parallel_loop(0, nvec, unroll=8, carry=zero16)
            def cnt16(vb, ptr_v):
                o = vb * 16
                s = rowbuf[pl.ds(o, 16)]
                msk = s >= tv
                pos = ptr_v + plsc.cumsum(msk.astype(i32)) - 1
                pos = jnp.minimum(pos, jnp.int32(136))
                plsc.store_scatter(ci, [pos], iota16 + o, mask=msk)
                return ptr_v + plsc.all_reduce_population_count(msk)
            for t in range(_C // 16):
                ii = ci[pl.ds(t * 16, 16)]
                valid = (iota16 + t * 16) < cnt16
                vals = plsc.load_gather(rowbuf, [ii], mask=valid)
                ob = r * _C + t * 16
                ov[pl.ds(ob, 16)] = jnp.where(
                    valid, vals, jnp.float32(-3.0))
                ox[pl.ds(ob, 16)] = plsc.load_gather(xv, [ii], mask=valid)
                oy[pl.ds(ob, 16)] = plsc.load_gather(yv, [ii], mask=valid)
                oz[pl.ds(ob, 16)] = plsc.load_gather(zv, [ii], mask=valid)

        def start(buf, sem, g):
            pltpu.async_copy(sim_hbm.at[pl.ds(g * n, n)], buf, sem)

        def drain(buf, sem):
            pltpu.make_async_copy(sim_hbm.at[pl.ds(0, n)], buf, sem).wait()

        for bb in range(rows_w // rb_rows):
            g0 = r0 + bb * rb_rows
            start(rowbuf0, sem0, g0)

            def pair(p, _, bb=bb, g0=g0):
                r = 2 * p
                drain(rowbuf0, sem0)
                start(rowbuf1, sem1, g0 + r + 1)
                process(rowbuf0, r, bb)
                drain(rowbuf1, sem1)
                start(rowbuf0, sem0,
                      g0 + jnp.minimum(r + 2, rb_rows - 1))
                process(rowbuf1, r + 1, bb)
                return 0

            lax.fori_loop(0, rb_rows // 2, pair, 0)
            drain(rowbuf0, sem0)
            base = g0 * _C
            pltpu.sync_copy(ov, cv_hbm.at[pl.ds(base, rb_rows * _C)])
            pltpu.sync_copy(ox, cx_hbm.at[pl.ds(base, rb_rows * _C)])
            pltpu.sync_copy(oy, cy_hbm.at[pl.ds(base, rb_rows * _C)])
            pltpu.sync_copy(oz, cz_hbm.at[pl.ds(base, rb_rows * _C)])

    return sc_compact


def _rank_body(cv_ref, cx_ref, cy_ref, cz_ref, pts_ref, dwt_ref, w_ref,
               bias_ref, out_ref, cv_s, sx_s, sy_s, sz_s):
    f32 = jnp.float32
    b = cv_ref.shape[0]
    lane_c = lax.broadcasted_iota(jnp.int32, (b, _C), 1)
    lane_k = lax.broadcasted_iota(jnp.int32, (b, _K), 1)
    cv_s[:, :] = cv_ref[:, :]
    cx = cx_ref[:, :]
    cy = cy_ref[:, :]
    cz = cz_ref[:, :]
    sx_s[:, :] = jnp.zeros((b, _K), f32)
    sy_s[:, :] = jnp.zeros((b, _K), f32)
    sz_s[:, :] = jnp.zeros((b, _K), f32)

    def body(j, _):
        cv = cv_s[:, :]
        m = jnp.max(cv, axis=1, keepdims=True)
        a = jnp.min(jnp.where(cv == m, lane_c, jnp.int32(_C)),
                    axis=1, keepdims=True)
        oh = lane_c == a
        vx = jnp.sum(jnp.where(oh, cx, 0.0), axis=1, keepdims=True)
        vy = jnp.sum(jnp.where(oh, cy, 0.0), axis=1, keepdims=True)
        vz = jnp.sum(jnp.where(oh, cz, 0.0), axis=1, keepdims=True)
        cv_s[:, :] = jnp.where(oh, jnp.float32(-3.0), cv)
        kj = lane_k == j
        sx_s[:, :] = jnp.where(kj, vx, sx_s[:, :])
        sy_s[:, :] = jnp.where(kj, vy, sy_s[:, :])
        sz_s[:, :] = jnp.where(kj, vz, sz_s[:, :])
        return 0

    lax.fori_loop(0, _K, body, 0)

    gx = sx_s[:, :]
    gy = sy_s[:, :]
    gz = sz_s[:, :]
    dx = gx - pts_ref[:, 0:1]
    dy = gy - pts_ref[:, 1:2]
    dz = gz - pts_ref[:, 2:3]
    dist = dx * dx + dy * dy + dz * dz + jnp.float32(1.0)
    acc = jnp.dot(gx * dwt_ref[0:1, :], w_ref[0], preferred_element_type=f32)
    acc = acc + jnp.dot(gy * dwt_ref[1:2, :], w_ref[1],
                        preferred_element_type=f32)
    acc = acc + jnp.dot(gz * dwt_ref[2:3, :], w_ref[2],
                        preferred_element_type=f32)
    acc = acc + jnp.dot(dist * dwt_ref[3:4, :], w_ref[3],
                        preferred_element_type=f32)
    out_ref[:, :] = acc + bias_ref[:, :]


def _rank_call(cv, cx, cy, cz, points, dwt, weight, bias2d):
    n = points.shape[0]
    grid = n // _BLK
    cout = weight.shape[2]
    return pl.pallas_call(
        _rank_body,
        grid=(grid,),
        in_specs=[
            pl.BlockSpec((_BLK, _C), lambda i: (i, 0)),
            pl.BlockSpec((_BLK, _C), lambda i: (i, 0)),
            pl.BlockSpec((_BLK, _C), lambda i: (i, 0)),
            pl.BlockSpec((_BLK, _C), lambda i: (i, 0)),
            pl.BlockSpec((_BLK, 3), lambda i: (i, 0)),
            pl.BlockSpec((4, _K), lambda i: (0, 0)),
            pl.BlockSpec((4, _K, cout), lambda i: (0, 0, 0)),
            pl.BlockSpec((1, cout), lambda i: (0, 0)),
        ],
        out_specs=pl.BlockSpec((_BLK, cout), lambda i: (i, 0)),
        out_shape=jax.ShapeDtypeStruct((n, cout), jnp.float32),
        scratch_shapes=[
            pltpu.VMEM((_BLK, _C), jnp.float32),
            pltpu.VMEM((_BLK, _K), jnp.float32),
            pltpu.VMEM((_BLK, _K), jnp.float32),
            pltpu.VMEM((_BLK, _K), jnp.float32),
        ],
    )(cv, cx, cy, cz, points, dwt, weight, bias2d)


def kernel(points, dw, weight, bias):
    n = points.shape[0]
    cout = weight.shape[2]
    norm = jnp.linalg.norm(points[:, :3], axis=-1, keepdims=True)
    xn = points[:, :3] / jnp.maximum(norm, 1e-12)
    xnt = xn.T

    sim, thr16 = _sim_thr_call(xn, xnt)

    px = jnp.asarray(points[:, 0], jnp.float32)
    py = jnp.asarray(points[:, 1], jnp.float32)
    pz = jnp.asarray(points[:, 2], jnp.float32)
    cv1, cx1, cy1, cz1 = _make_sc_compact(n)(
        sim.reshape(-1), thr16.reshape(-1), px, py, pz)
    cv = cv1.reshape(n, _C)
    cx = cx1.reshape(n, _C)
    cy = cy1.reshape(n, _C)
    cz = cz1.reshape(n, _C)

    return _rank_call(cv, cx, cy, cz, points, dw.T, weight,
                      bias.reshape(1, cout))


# speedup vs baseline: 1.8728x; 1.1002x over previous
import functools

import jax
import jax.numpy as jnp
from jax import lax
from jax.experimental import pallas as pl
from jax.experimental.pallas import tpu as pltpu
from jax.experimental.pallas import tpu_sc as plsc

_N = 8192
_K = 81
_BLK = 256
_C = 128
_BIS = 12


def _sim_thr_body(xnb_ref, xnt_ref, sim_ref, thr_ref):
    b, n = sim_ref.shape
    row0 = pl.program_id(0) * b
    col_ids = lax.broadcasted_iota(jnp.int32, (b, n), 1)
    row_ids = row0 + lax.broadcasted_iota(jnp.int32, (b, n), 0)
    xb16 = xnb_ref[:, :].astype(jnp.bfloat16)
    yt16 = xnt_ref[:, :].astype(jnp.bfloat16)
    sim = jnp.dot(xb16, yt16, preferred_element_type=jnp.float32)
    sim = jnp.where(col_ids == row_ids, jnp.float32(2.0), sim)
    sim_ref[:, :] = sim

    lo0 = jnp.full((b, 1), -1.0, jnp.float32)
    hi0 = jnp.full((b, 1), 1.0, jnp.float32)

    def bis(_, c):
        lo, hi = c
        mid = jnp.float32(0.5) * (lo + hi)
        cnt = jnp.sum(jnp.where(sim_ref[:, :] >= mid, 1.0, 0.0),
                      axis=1, keepdims=True)
        p = cnt >= jnp.float32(_K)
        return (jnp.where(p, mid, lo), jnp.where(p, hi, mid))

    lo, hi = lax.fori_loop(0, _BIS, bis, (lo0, hi0))
    thr_ref[:, :] = jnp.broadcast_to(lo, (b, 16))


def _sim_thr_call(xn, xnt):
    n = xn.shape[0]
    grid = n // _BLK
    return pl.pallas_call(
        _sim_thr_body,
        grid=(grid,),
        in_specs=[
            pl.BlockSpec((_BLK, 3), lambda i: (i, 0)),
            pl.BlockSpec((3, n), lambda i: (0, 0)),
        ],
        out_specs=[
            pl.BlockSpec((_BLK, n), lambda i: (i, 0)),
            pl.BlockSpec((_BLK, 16), lambda i: (i, 0)),
        ],
        out_shape=[
            jax.ShapeDtypeStruct((n, n), jnp.float32),
            jax.ShapeDtypeStruct((n, 16), jnp.float32),
        ],
    )(xn, xnt)


def _make_sc_compact(n):
    info = plsc.get_sparse_core_info()
    nw = info.num_cores * info.num_subcores
    rows_w = n // nw
    rb_rows = 64
    nvec = n // 16
    i32, f32 = jnp.int32, jnp.float32
    mesh = plsc.VectorSubcoreMesh(core_axis_name="c", subcore_axis_name="s")

    @functools.partial(
        pl.kernel,
        mesh=mesh,
        out_type=(jax.ShapeDtypeStruct((n * _C,), f32),) * 4,
        compiler_params=pltpu.CompilerParams(needs_layout_passes=False),
        scratch_types=[
            pltpu.VMEM((n,), f32),
            pltpu.VMEM((n,), f32),
            pltpu.VMEM((n,), f32),
            pltpu.VMEM((n,), f32),
            pltpu.VMEM((n,), f32),
            pltpu.VMEM((rows_w * 16,), f32),
            pltpu.VMEM((144,), i32),
            pltpu.SemaphoreType.DMA,
            pltpu.SemaphoreType.DMA,
            pltpu.VMEM((rb_rows * _C,), f32),
            pltpu.VMEM((rb_rows * _C,), f32),
            pltpu.VMEM((rb_rows * _C,), f32),
            pltpu.VMEM((rb_rows * _C,), f32),
        ],
    )
    def sc_compact(sim_hbm, thr_hbm, x_hbm, y_hbm, z_hbm,
                   cv_hbm, cx_hbm, cy_hbm, cz_hbm,
                   xv, yv, zv, rowbuf0, rowbuf1, thrv, ci, sem0, sem1,
                   ov, ox, oy, oz):
        wid = lax.axis_index("s") * info.num_cores + lax.axis_index("c")
        r0 = wid * rows_w
        pltpu.sync_copy(x_hbm, xv)
        pltpu.sync_copy(y_hbm, yv)
        pltpu.sync_copy(z_hbm, zv)
        pltpu.sync_copy(thr_hbm.at[pl.ds(r0 * 16, rows_w * 16)], thrv)
        iota16 = lax.iota(i32, 16)
        zero16 = jnp.zeros((16,), i32)

        def process(rowbuf, r, bb):
            tv = thrv[pl.ds((bb * rb_rows + r) * 16, 16)]

            @plsc.parallel_loop(0, nvec, unroll=8, carry=zero16)
            def cnt16(vb, ptr_v):
                o = vb * 16
                s = rowbuf[pl.ds(o, 16)]
                msk = s >= tv
                pos = ptr_v + plsc.cumsum(msk.astype(i32)) - 1
                pos = jnp.minimum(pos, jnp.int32(136))
                plsc.store_scatter(ci, [pos], iota16 + o, mask=msk)
                return ptr_v + plsc.all_reduce_population_count(msk)
            for t in range(_C // 16):
                ii = ci[pl.ds(t * 16, 16)]
                valid = (iota16 + t * 16) < cnt16
                vals = plsc.load_gather(rowbuf, [ii], mask=valid)
                ob = r * _C + t * 16
                ov[pl.ds(ob, 16)] = jnp.where(
                    valid, vals, jnp.float32(-3.0))
                ox[pl.ds(ob, 16)] = plsc.load_gather(xv, [ii], mask=valid)
                oy[pl.ds(ob, 16)] = plsc.load_gather(yv, [ii], mask=valid)
                oz[pl.ds(ob, 16)] = plsc.load_gather(zv, [ii], mask=valid)

        def start(buf, sem, g):
            pltpu.async_copy(sim_hbm.at[g], buf, sem)

        def drain(buf, sem):
            pltpu.make_async_copy(sim_hbm.at[0], buf, sem).wait()

        for bb in range(rows_w // rb_rows):
            g0 = r0 + bb * rb_rows
            start(rowbuf0, sem0, g0)

            def pair(p, _, bb=bb, g0=g0):
                r = 2 * p
                drain(rowbuf0, sem0)
                start(rowbuf1, sem1, g0 + r + 1)
                process(rowbuf0, r, bb)
                drain(rowbuf1, sem1)
                start(rowbuf0, sem0,
                      g0 + jnp.minimum(r + 2, rb_rows - 1))
                process(rowbuf1, r + 1, bb)
                return 0

            lax.fori_loop(0, rb_rows // 2, pair, 0)
            drain(rowbuf0, sem0)
            base = g0 * _C
            pltpu.sync_copy(ov, cv_hbm.at[pl.ds(base, rb_rows * _C)])
            pltpu.sync_copy(ox, cx_hbm.at[pl.ds(base, rb_rows * _C)])
            pltpu.sync_copy(oy, cy_hbm.at[pl.ds(base, rb_rows * _C)])
            pltpu.sync_copy(oz, cz_hbm.at[pl.ds(base, rb_rows * _C)])

    return sc_compact


def _rank_body(cv_ref, cx_ref, cy_ref, cz_ref, pts_ref, dwt_ref, w_ref,
               bias_ref, out_ref, cv_s, sx_s, sy_s, sz_s):
    f32 = jnp.float32
    b = cv_ref.shape[0]
    lane_c = lax.broadcasted_iota(jnp.int32, (b, _C), 1)
    lane_k = lax.broadcasted_iota(jnp.int32, (b, _K), 1)
    cv_s[:, :] = cv_ref[:, :]
    cx = cx_ref[:, :]
    cy = cy_ref[:, :]
    cz = cz_ref[:, :]
    sx_s[:, :] = jnp.zeros((b, _K), f32)
    sy_s[:, :] = jnp.zeros((b, _K), f32)
    sz_s[:, :] = jnp.zeros((b, _K), f32)

    def body(j, _):
        cv = cv_s[:, :]
        m = jnp.max(cv, axis=1, keepdims=True)
        a = jnp.min(jnp.where(cv == m, lane_c, jnp.int32(_C)),
                    axis=1, keepdims=True)
        oh = lane_c == a
        vx = jnp.sum(jnp.where(oh, cx, 0.0), axis=1, keepdims=True)
        vy = jnp.sum(jnp.where(oh, cy, 0.0), axis=1, keepdims=True)
        vz = jnp.sum(jnp.where(oh, cz, 0.0), axis=1, keepdims=True)
        cv_s[:, :] = jnp.where(oh, jnp.float32(-3.0), cv)
        kj = lane_k == j
        sx_s[:, :] = jnp.where(kj, vx, sx_s[:, :])
        sy_s[:, :] = jnp.where(kj, vy, sy_s[:, :])
        sz_s[:, :] = jnp.where(kj, vz, sz_s[:, :])
        return 0

    lax.fori_loop(0, _K, body, 0)

    gx = sx_s[:, :]
    gy = sy_s[:, :]
    gz = sz_s[:, :]
    dx = gx - pts_ref[:, 0:1]
    dy = gy - pts_ref[:, 1:2]
    dz = gz - pts_ref[:, 2:3]
    dist = dx * dx + dy * dy + dz * dz + jnp.float32(1.0)
    acc = jnp.dot(gx * dwt_ref[0:1, :], w_ref[0], preferred_element_type=f32)
    acc = acc + jnp.dot(gy * dwt_ref[1:2, :], w_ref[1],
                        preferred_element_type=f32)
    acc = acc + jnp.dot(gz * dwt_ref[2:3, :], w_ref[2],
                        preferred_element_type=f32)
    acc = acc + jnp.dot(dist * dwt_ref[3:4, :], w_ref[3],
                        preferred_element_type=f32)
    out_ref[:, :] = acc + bias_ref[:, :]


def _rank_call(cv, cx, cy, cz, points, dwt, weight, bias2d):
    n = points.shape[0]
    grid = n // _BLK
    cout = weight.shape[2]
    return pl.pallas_call(
        _rank_body,
        grid=(grid,),
        in_specs=[
            pl.BlockSpec((_BLK, _C), lambda i: (i, 0)),
            pl.BlockSpec((_BLK, _C), lambda i: (i, 0)),
            pl.BlockSpec((_BLK, _C), lambda i: (i, 0)),
            pl.BlockSpec((_BLK, _C), lambda i: (i, 0)),
            pl.BlockSpec((_BLK, 3), lambda i: (i, 0)),
            pl.BlockSpec((4, _K), lambda i: (0, 0)),
            pl.BlockSpec((4, _K, cout), lambda i: (0, 0, 0)),
            pl.BlockSpec((1, cout), lambda i: (0, 0)),
        ],
        out_specs=pl.BlockSpec((_BLK, cout), lambda i: (i, 0)),
        out_shape=jax.ShapeDtypeStruct((n, cout), jnp.float32),
        scratch_shapes=[
            pltpu.VMEM((_BLK, _C), jnp.float32),
            pltpu.VMEM((_BLK, _K), jnp.float32),
            pltpu.VMEM((_BLK, _K), jnp.float32),
            pltpu.VMEM((_BLK, _K), jnp.float32),
        ],
    )(cv, cx, cy, cz, points, dwt, weight, bias2d)


def kernel(points, dw, weight, bias):
    n = points.shape[0]
    cout = weight.shape[2]
    norm = jnp.linalg.norm(points[:, :3], axis=-1, keepdims=True)
    xn = points[:, :3] / jnp.maximum(norm, 1e-12)
    xnt = xn.T

    sim, thr16 = _sim_thr_call(xn, xnt)

    px = jnp.asarray(points[:, 0], jnp.float32)
    py = jnp.asarray(points[:, 1], jnp.float32)
    pz = jnp.asarray(points[:, 2], jnp.float32)
    cv1, cx1, cy1, cz1 = _make_sc_compact(n)(
        sim, thr16.reshape(-1), px, py, pz)
    cv = cv1.reshape(n, _C)
    cx = cx1.reshape(n, _C)
    cy = cy1.reshape(n, _C)
    cz = cz1.reshape(n, _C)

    return _rank_call(cv, cx, cy, cz, points, dw.T, weight,
                      bias.reshape(1, cout))


# speedup vs baseline: 2.2702x; 1.2122x over previous
import functools

import jax
import jax.numpy as jnp
from jax import lax
from jax.experimental import pallas as pl
from jax.experimental.pallas import tpu as pltpu
from jax.experimental.pallas import tpu_sc as plsc

_N = 8192
_K = 81
_BLK = 256
_C = 128
_BIS = 11


def _sim_thr_body(xnb_ref, xnt_ref, sim_ref, thr_ref, *, row_base):
    b, n = sim_ref.shape
    row0 = row_base + pl.program_id(0) * b
    col_ids = lax.broadcasted_iota(jnp.int32, (b, n), 1)
    row_ids = row0 + lax.broadcasted_iota(jnp.int32, (b, n), 0)
    xb16 = xnb_ref[:, :].astype(jnp.bfloat16)
    yt16 = xnt_ref[:, :].astype(jnp.bfloat16)
    sim = jnp.dot(xb16, yt16, preferred_element_type=jnp.float32)
    sim = jnp.where(col_ids == row_ids, jnp.float32(2.0), sim)
    sim_ref[:, :] = sim

    lo0 = jnp.full((b, 1), -1.0, jnp.float32)
    hi0 = jnp.full((b, 1), 1.0, jnp.float32)

    def bis(_, c):
        lo, hi = c
        mid = jnp.float32(0.5) * (lo + hi)
        cnt = jnp.sum(jnp.where(sim_ref[:, :] >= mid, 1.0, 0.0),
                      axis=1, keepdims=True)
        p = cnt >= jnp.float32(_K)
        return (jnp.where(p, mid, lo), jnp.where(p, hi, mid))

    lo, hi = lax.fori_loop(0, _BIS, bis, (lo0, hi0))
    thr_ref[:, :] = jnp.broadcast_to(lo, (b, 16))


def _sim_thr_call(xn_part, xnt, row_base):
    nr = xn_part.shape[0]
    n = xnt.shape[1]
    grid = nr // _BLK
    return pl.pallas_call(
        functools.partial(_sim_thr_body, row_base=row_base),
        grid=(grid,),
        in_specs=[
            pl.BlockSpec((_BLK, 3), lambda i: (i, 0)),
            pl.BlockSpec((3, n), lambda i: (0, 0)),
        ],
        out_specs=[
            pl.BlockSpec((_BLK, n), lambda i: (i, 0)),
            pl.BlockSpec((_BLK, 16), lambda i: (i, 0)),
        ],
        out_shape=[
            jax.ShapeDtypeStruct((nr, n), jnp.float32),
            jax.ShapeDtypeStruct((nr, 16), jnp.float32),
        ],
    )(xn_part, xnt)


def _make_sc_compact(n, nr):
    info = plsc.get_sparse_core_info()
    nw = info.num_cores * info.num_subcores
    rows_w = nr // nw
    rb_rows = 64
    nvec = n // 16
    i32, f32 = jnp.int32, jnp.float32
    mesh = plsc.VectorSubcoreMesh(core_axis_name="c", subcore_axis_name="s")

    @functools.partial(
        pl.kernel,
        mesh=mesh,
        out_type=(jax.ShapeDtypeStruct((nr * _C,), f32),) * 4,
        compiler_params=pltpu.CompilerParams(needs_layout_passes=False),
        scratch_types=[
            pltpu.VMEM((n,), f32),
            pltpu.VMEM((n,), f32),
            pltpu.VMEM((n,), f32),
            pltpu.VMEM((n,), f32),
            pltpu.VMEM((n,), f32),
            pltpu.VMEM((rows_w * 16,), f32),
            pltpu.VMEM((144,), i32),
            pltpu.SemaphoreType.DMA,
            pltpu.SemaphoreType.DMA,
            pltpu.VMEM((rb_rows * _C,), f32),
            pltpu.VMEM((rb_rows * _C,), f32),
            pltpu.VMEM((rb_rows * _C,), f32),
            pltpu.VMEM((rb_rows * _C,), f32),
        ],
    )
    def sc_compact(sim_hbm, thr_hbm, x_hbm, y_hbm, z_hbm,
                   cv_hbm, cx_hbm, cy_hbm, cz_hbm,
                   xv, yv, zv, rowbuf0, rowbuf1, thrv, ci, sem0, sem1,
                   ov, ox, oy, oz):
        wid = lax.axis_index("s") * info.num_cores + lax.axis_index("c")
        r0 = wid * rows_w
        pltpu.sync_copy(x_hbm, xv)
        pltpu.sync_copy(y_hbm, yv)
        pltpu.sync_copy(z_hbm, zv)
        pltpu.sync_copy(thr_hbm.at[pl.ds(r0 * 16, rows_w * 16)], thrv)
        iota16 = lax.iota(i32, 16)
        zero16 = jnp.zeros((16,), i32)

        def process(rowbuf, r, bb):
            tv = thrv[pl.ds((bb * rb_rows + r) * 16, 16)]

            @plsc.parallel_loop(0, nvec, unroll=8, carry=zero16)
            def cnt16(vb, ptr_v):
                o = vb * 16
                s = rowbuf[pl.ds(o, 16)]
                msk = s >= tv
                pos = ptr_v + plsc.cumsum(msk.astype(i32)) - 1
                pos = jnp.minimum(pos, jnp.int32(136))
                plsc.store_scatter(ci, [pos], iota16 + o, mask=msk)
                return ptr_v + plsc.all_reduce_population_count(msk)
            for t in range(_C // 16):
                ii = ci[pl.ds(t * 16, 16)]
                valid = (iota16 + t * 16) < cnt16
                vals = plsc.load_gather(rowbuf, [ii], mask=valid)
                ob = r * _C + t * 16
                ov[pl.ds(ob, 16)] = jnp.where(
                    valid, vals, jnp.float32(-3.0))
                ox[pl.ds(ob, 16)] = plsc.load_gather(xv, [ii], mask=valid)
                oy[pl.ds(ob, 16)] = plsc.load_gather(yv, [ii], mask=valid)
                oz[pl.ds(ob, 16)] = plsc.load_gather(zv, [ii], mask=valid)

        def start(buf, sem, g):
            pltpu.async_copy(sim_hbm.at[g], buf, sem)

        def drain(buf, sem):
            pltpu.make_async_copy(sim_hbm.at[0], buf, sem).wait()

        for bb in range(rows_w // rb_rows):
            g0 = r0 + bb * rb_rows
            start(rowbuf0, sem0, g0)

            def pair(p, _, bb=bb, g0=g0):
                r = 2 * p
                drain(rowbuf0, sem0)
                start(rowbuf1, sem1, g0 + r + 1)
                process(rowbuf0, r, bb)
                drain(rowbuf1, sem1)
                start(rowbuf0, sem0,
                      g0 + jnp.minimum(r + 2, rb_rows - 1))
                process(rowbuf1, r + 1, bb)
                return 0

            lax.fori_loop(0, rb_rows // 2, pair, 0)
            drain(rowbuf0, sem0)
            base = g0 * _C
            pltpu.sync_copy(ov, cv_hbm.at[pl.ds(base, rb_rows * _C)])
            pltpu.sync_copy(ox, cx_hbm.at[pl.ds(base, rb_rows * _C)])
            pltpu.sync_copy(oy, cy_hbm.at[pl.ds(base, rb_rows * _C)])
            pltpu.sync_copy(oz, cz_hbm.at[pl.ds(base, rb_rows * _C)])

    return sc_compact


def _rank_body(cv_ref, cx_ref, cy_ref, cz_ref, pts_ref, dwt_ref, w_ref,
               bias_ref, out_ref, cv_s, sx_s, sy_s, sz_s):
    f32 = jnp.float32
    b = cv_ref.shape[0]
    lane_c = lax.broadcasted_iota(jnp.int32, (b, _C), 1)
    lane_k = lax.broadcasted_iota(jnp.int32, (b, _K), 1)
    cv_s[:, :] = cv_ref[:, :]
    cx = cx_ref[:, :]
    cy = cy_ref[:, :]
    cz = cz_ref[:, :]
    sx_s[:, :] = jnp.zeros((b, _K), f32)
    sy_s[:, :] = jnp.zeros((b, _K), f32)
    sz_s[:, :] = jnp.zeros((b, _K), f32)

    def body(j, _):
        cv = cv_s[:, :]
        m = jnp.max(cv, axis=1, keepdims=True)
        a = jnp.min(jnp.where(cv == m, lane_c, jnp.int32(_C)),
                    axis=1, keepdims=True)
        oh = lane_c == a
        vx = jnp.sum(jnp.where(oh, cx, 0.0), axis=1, keepdims=True)
        vy = jnp.sum(jnp.where(oh, cy, 0.0), axis=1, keepdims=True)
        vz = jnp.sum(jnp.where(oh, cz, 0.0), axis=1, keepdims=True)
        cv_s[:, :] = jnp.where(oh, jnp.float32(-3.0), cv)
        kj = lane_k == j
        sx_s[:, :] = jnp.where(kj, vx, sx_s[:, :])
        sy_s[:, :] = jnp.where(kj, vy, sy_s[:, :])
        sz_s[:, :] = jnp.where(kj, vz, sz_s[:, :])
        return 0

    lax.fori_loop(0, _K, body, 0)

    gx = sx_s[:, :]
    gy = sy_s[:, :]
    gz = sz_s[:, :]
    dx = gx - pts_ref[:, 0:1]
    dy = gy - pts_ref[:, 1:2]
    dz = gz - pts_ref[:, 2:3]
    dist = dx * dx + dy * dy + dz * dz + jnp.float32(1.0)
    acc = jnp.dot(gx * dwt_ref[0:1, :], w_ref[0], preferred_element_type=f32)
    acc = acc + jnp.dot(gy * dwt_ref[1:2, :], w_ref[1],
                        preferred_element_type=f32)
    acc = acc + jnp.dot(gz * dwt_ref[2:3, :], w_ref[2],
                        preferred_element_type=f32)
    acc = acc + jnp.dot(dist * dwt_ref[3:4, :], w_ref[3],
                        preferred_element_type=f32)
    out_ref[:, :] = acc + bias_ref[:, :]


def _rank_call(cv, cx, cy, cz, points, dwt, weight, bias2d):
    n = points.shape[0]
    grid = n // _BLK
    cout = weight.shape[2]
    return pl.pallas_call(
        _rank_body,
        grid=(grid,),
        in_specs=[
            pl.BlockSpec((_BLK, _C), lambda i: (i, 0)),
            pl.BlockSpec((_BLK, _C), lambda i: (i, 0)),
            pl.BlockSpec((_BLK, _C), lambda i: (i, 0)),
            pl.BlockSpec((_BLK, _C), lambda i: (i, 0)),
            pl.BlockSpec((_BLK, 3), lambda i: (i, 0)),
            pl.BlockSpec((4, _K), lambda i: (0, 0)),
            pl.BlockSpec((4, _K, cout), lambda i: (0, 0, 0)),
            pl.BlockSpec((1, cout), lambda i: (0, 0)),
        ],
        out_specs=pl.BlockSpec((_BLK, cout), lambda i: (i, 0)),
        out_shape=jax.ShapeDtypeStruct((n, cout), jnp.float32),
        scratch_shapes=[
            pltpu.VMEM((_BLK, _C), jnp.float32),
            pltpu.VMEM((_BLK, _K), jnp.float32),
            pltpu.VMEM((_BLK, _K), jnp.float32),
            pltpu.VMEM((_BLK, _K), jnp.float32),
        ],
    )(cv, cx, cy, cz, points, dwt, weight, bias2d)


def kernel(points, dw, weight, bias):
    n = points.shape[0]
    cout = weight.shape[2]
    norm = jnp.linalg.norm(points[:, :3], axis=-1, keepdims=True)
    xn = points[:, :3] / jnp.maximum(norm, 1e-12)
    xnt = xn.T

    px = jnp.asarray(points[:, 0], jnp.float32)
    py = jnp.asarray(points[:, 1], jnp.float32)
    pz = jnp.asarray(points[:, 2], jnp.float32)

    nh = n // 2
    sc_compact = _make_sc_compact(n, nh)
    outs = []
    for off in (0, nh):
        sim, thr16 = _sim_thr_call(
            lax.slice_in_dim(xn, off, off + nh, axis=0), xnt, off)
        cv1, cx1, cy1, cz1 = sc_compact(sim, thr16.reshape(-1), px, py, pz)
        outs.append(_rank_call(
            cv1.reshape(nh, _C), cx1.reshape(nh, _C),
            cy1.reshape(nh, _C), cz1.reshape(nh, _C),
            lax.slice_in_dim(points, off, off + nh, axis=0),
            dw.T, weight, bias.reshape(1, cout)))
    return jnp.concatenate(outs, axis=0)
